# baseline (speedup 1.0000x reference)
import functools

import jax
import jax.numpy as jnp
from jax import lax
from jax.experimental import pallas as pl
from jax.experimental.pallas import tpu as pltpu
from jax.experimental.pallas import tpu_sc as plsc

NUM_ENTITIES = 1000000
NUM_RELATIONS = 1000
EMBED_DIM = 64
BATCH = 16384

NC = 2
NS = 16
LANES = 16
NW = NC * NS
B_PER_W = BATCH // NW
CHUNK = 128
NCHUNK = B_PER_W // CHUNK
GROUPS = CHUNK // LANES
NSLICE = EMBED_DIM // LANES
ROWS_PER_IT = 16


def _scores_sc(heads, relations, tails, entity_weight, relation_weight):
    mesh = plsc.VectorSubcoreMesh(core_axis_name="c", subcore_axis_name="s")

    @functools.partial(
        pl.kernel,
        out_type=jax.ShapeDtypeStruct((BATCH,), jnp.float32),
        mesh=mesh,
        compiler_params=pltpu.CompilerParams(needs_layout_passes=False),
        scratch_types=[
            pltpu.VMEM((B_PER_W,), jnp.int32),
            pltpu.VMEM((B_PER_W,), jnp.int32),
            pltpu.VMEM((B_PER_W,), jnp.int32),
            pltpu.VMEM((2, CHUNK, EMBED_DIM), jnp.float32),
            pltpu.VMEM((2, CHUNK, EMBED_DIM), jnp.float32),
            pltpu.VMEM((2, CHUNK, EMBED_DIM), jnp.float32),
            pltpu.VMEM((B_PER_W,), jnp.float32),
            pltpu.VMEM((LANES * LANES,), jnp.float32),
            pltpu.SemaphoreType.DMA,
            pltpu.SemaphoreType.DMA,
        ],
    )
    def k(heads_hbm, rels_hbm, tails_hbm, ent_hbm, relw_hbm, out_hbm,
          hidx_v, tidx_v, ridx_v, h_v, t_v, r_v, out_v, pt_v, sem0, sem1):
        wid = lax.axis_index("s") * NC + lax.axis_index("c")
        base = wid * B_PER_W

        pltpu.sync_copy(heads_hbm.at[pl.ds(base, B_PER_W)], hidx_v)
        pltpu.sync_copy(tails_hbm.at[pl.ds(base, B_PER_W)], tidx_v)
        pltpu.sync_copy(rels_hbm.at[pl.ds(base, B_PER_W)], ridx_v)

        sems = (sem0, sem1)

        def fire_chunk(c):
            p = c % 2
            sem = sems[p]

            def issue(it, _):
                off = c * CHUNK + it * ROWS_PER_IT
                hvec = hidx_v[pl.ds(off, ROWS_PER_IT)]
                tvec = tidx_v[pl.ds(off, ROWS_PER_IT)]
                rvec = ridx_v[pl.ds(off, ROWS_PER_IT)]
                for u in range(ROWS_PER_IT):
                    j = it * ROWS_PER_IT + u
                    pltpu.make_async_copy(
                        ent_hbm.at[pl.ds(hvec[u], 1)],
                        h_v.at[p, pl.ds(j, 1)], sem).start()
                    pltpu.make_async_copy(
                        ent_hbm.at[pl.ds(tvec[u], 1)],
                        t_v.at[p, pl.ds(j, 1)], sem).start()
                    pltpu.make_async_copy(
                        relw_hbm.at[pl.ds(rvec[u], 1)],
                        r_v.at[p, pl.ds(j, 1)], sem).start()
                return 0

            lax.fori_loop(0, CHUNK // ROWS_PER_IT, issue, 0)

        def drain_chunk(c):
            p = c % 2
            sem = sems[p]
            pltpu.make_async_copy(
                ent_hbm.at[pl.ds(0, CHUNK)], h_v.at[p], sem).wait()
            pltpu.make_async_copy(
                ent_hbm.at[pl.ds(0, CHUNK)], t_v.at[p], sem).wait()
            pltpu.make_async_copy(
                ent_hbm.at[pl.ds(0, CHUNK)], r_v.at[p], sem).wait()

        lanes = lax.iota(jnp.int32, 16)

        def compute_chunk(c):
            p = c % 2

            def group_body(g, _):
                for j in range(LANES):
                    i = g * LANES + j
                    terms = []
                    for s in range(NSLICE):
                        sl = pl.ds(s * LANES, LANES)
                        terms.append(jnp.abs(
                            h_v[p, i, sl] + r_v[p, i, sl] - t_v[p, i, sl]))
                    part = (terms[0] + terms[1]) + (terms[2] + terms[3])
                    plsc.store_scatter(pt_v, [lanes * LANES + j], part)
                cols = [pt_v[pl.ds(l * LANES, LANES)] for l in range(LANES)]
                while len(cols) > 1:
                    cols = [cols[2 * m] + cols[2 * m + 1]
                            for m in range(len(cols) // 2)]
                out_v[pl.ds(c * CHUNK + g * LANES, LANES)] = cols[0]
                return 0

            lax.fori_loop(0, GROUPS, group_body, 0)

        fire_chunk(0)
        for c in range(NCHUNK):
            if c + 1 < NCHUNK:
                fire_chunk(c + 1)
            drain_chunk(c)
            compute_chunk(c)

        pltpu.sync_copy(out_v, out_hbm.at[pl.ds(base, B_PER_W)])

    return k(heads, relations, tails, entity_weight, relation_weight)


def kernel(heads, relations, tails, entity_weight, relation_weight):
    return _scores_sc(heads.astype(jnp.int32), relations.astype(jnp.int32),
                      tails.astype(jnp.int32), entity_weight, relation_weight)
